# baseline (speedup 1.0000x reference)
import functools

import jax
import jax.numpy as jnp
from jax import lax
from jax.experimental import pallas as pl
from jax.experimental.pallas import tpu as pltpu
from jax.experimental.pallas import tpu_sc as plsc

_F, _V, _D_NAME, _D_VAL = 100, 100, 16, 16
_OUT_D = _D_NAME + _D_VAL
_ROW = _F * _OUT_D
_BBLK = 512
_FPAD = 256


def _make_sc_gather():
    info = plsc.get_sparse_core_info()
    nc, ns = info.num_cores, info.num_subcores
    nw = nc * ns
    b_per_w = _FPAD // nw
    mesh = plsc.VectorSubcoreMesh(core_axis_name="c", subcore_axis_name="s")

    @functools.partial(
        pl.kernel, mesh=mesh,
        out_type=jax.ShapeDtypeStruct((_FPAD, 128), jnp.float32),
        scratch_types=[
            pltpu.VMEM((b_per_w,), jnp.int32),
            pltpu.VMEM((b_per_w, 128), jnp.float32),
            pltpu.SemaphoreType.DMA,
        ],
    )
    def gather(table_hbm, idx_hbm, out_hbm, idx_v, rows_v, sem):
        wid = lax.axis_index("s") * nc + lax.axis_index("c")
        base = wid * b_per_w
        pltpu.sync_copy(idx_hbm.at[pl.ds(base, b_per_w)], idx_v)
        pltpu.async_copy(table_hbm.at[idx_v], rows_v, sem).wait()
        pltpu.sync_copy(rows_v, out_hbm.at[pl.ds(base, b_per_w)])

    return gather


def _emb_kernel(fv_ref, emb_ref, w_ref, b_ref, out_ref):
    name_emb = emb_ref[...]
    bias = jnp.broadcast_to(b_ref[...], (_F, _D_VAL))
    name_part = jnp.concatenate([name_emb, bias], axis=1)

    jio = lax.broadcasted_iota(jnp.int32, (_F, _ROW), 1)
    fio = lax.broadcasted_iota(jnp.int32, (_F, _ROW), 0)
    e_mat = ((jio // _OUT_D) == fio).astype(jnp.float32)
    jio2 = lax.broadcasted_iota(jnp.int32, (_OUT_D, _ROW), 1)
    cio = lax.broadcasted_iota(jnp.int32, (_OUT_D, _ROW), 0)
    g_mat = ((jio2 % _OUT_D) == cio).astype(jnp.float32)

    np_exp = lax.dot_general(
        name_part, e_mat, (((0,), (0,)), ((), ())),
        preferred_element_type=jnp.float32)
    name_row = jnp.sum(g_mat * np_exp, axis=0, keepdims=True)

    scale = jnp.concatenate(
        [jnp.zeros((1, _D_NAME), jnp.float32), w_ref[...].T], axis=1)
    scale_row = lax.dot_general(
        scale, g_mat, (((1,), (0,)), ((), ())),
        preferred_element_type=jnp.float32)
    a_mat = e_mat * scale_row

    out_ref[...] = lax.dot_general(
        fv_ref[...], a_mat, (((1,), (0,)), ((), ())),
        preferred_element_type=jnp.float32) + name_row


def kernel(feature_values, name_table, W, b, name_indices):
    batch = feature_values.shape[0]
    b2 = b.reshape(1, _D_VAL)
    idx_pad = jnp.zeros((_FPAD,), jnp.int32).at[:_F].set(
        name_indices.astype(jnp.int32))
    table128 = jnp.pad(name_table, ((0, 0), (0, 128 - _D_NAME)))
    name_emb = _make_sc_gather()(table128, idx_pad)[:_F, :_D_NAME]
    out = pl.pallas_call(
        _emb_kernel,
        grid=(batch // _BBLK,),
        in_specs=[
            pl.BlockSpec((_BBLK, _F), lambda i: (i, 0)),
            pl.BlockSpec((_F, _D_NAME), lambda i: (0, 0)),
            pl.BlockSpec((_D_VAL, 1), lambda i: (0, 0)),
            pl.BlockSpec((1, _D_VAL), lambda i: (0, 0)),
        ],
        out_specs=pl.BlockSpec((_BBLK, _ROW), lambda i: (i, 0)),
        out_shape=jax.ShapeDtypeStruct((batch, _ROW), jnp.float32),
    )(feature_values, name_emb, W, b2)
    return out.reshape(batch, _F, _OUT_D)


# speedup vs baseline: 1.0042x; 1.0042x over previous
import functools

import jax
import jax.numpy as jnp
from jax import lax
from jax.experimental import pallas as pl
from jax.experimental.pallas import tpu as pltpu
from jax.experimental.pallas import tpu_sc as plsc

_F, _V, _D_NAME, _D_VAL = 100, 100, 16, 16
_OUT_D = _D_NAME + _D_VAL
_ROW = _F * _OUT_D
_BBLK = 512
_FPAD = 256


def _make_sc_gather():
    info = plsc.get_sparse_core_info()
    nc, ns = info.num_cores, info.num_subcores
    nw = nc * ns
    b_per_w = _FPAD // nw
    mesh = plsc.VectorSubcoreMesh(core_axis_name="c", subcore_axis_name="s")

    @functools.partial(
        pl.kernel, mesh=mesh,
        out_type=jax.ShapeDtypeStruct((_FPAD, 128), jnp.float32),
        scratch_types=[
            pltpu.VMEM((b_per_w,), jnp.int32),
            pltpu.VMEM((b_per_w, 128), jnp.float32),
            pltpu.SemaphoreType.DMA,
        ],
    )
    def gather(table_hbm, idx_hbm, out_hbm, idx_v, rows_v, sem):
        wid = lax.axis_index("s") * nc + lax.axis_index("c")
        base = wid * b_per_w
        pltpu.sync_copy(idx_hbm.at[pl.ds(base, b_per_w)], idx_v)
        pltpu.async_copy(table_hbm.at[idx_v], rows_v, sem).wait()
        pltpu.sync_copy(rows_v, out_hbm.at[pl.ds(base, b_per_w)])

    return gather


def _emb_kernel(fv_ref, emb_ref, w_ref, b_ref, out_ref):
    name_emb = emb_ref[...][:_F, :_D_NAME]
    bias = jnp.broadcast_to(b_ref[...], (_F, _D_VAL))
    name_part = jnp.concatenate([name_emb, bias], axis=1)

    jio = lax.broadcasted_iota(jnp.int32, (_F, _ROW), 1)
    fio = lax.broadcasted_iota(jnp.int32, (_F, _ROW), 0)
    e_mat = ((jio // _OUT_D) == fio).astype(jnp.float32)
    jio2 = lax.broadcasted_iota(jnp.int32, (_OUT_D, _ROW), 1)
    cio = lax.broadcasted_iota(jnp.int32, (_OUT_D, _ROW), 0)
    g_mat = ((jio2 % _OUT_D) == cio).astype(jnp.float32)

    np_exp = lax.dot_general(
        name_part, e_mat, (((0,), (0,)), ((), ())),
        preferred_element_type=jnp.float32)
    name_row = jnp.sum(g_mat * np_exp, axis=0, keepdims=True)

    scale = jnp.concatenate(
        [jnp.zeros((1, _D_NAME), jnp.float32), w_ref[...].T], axis=1)
    scale_row = lax.dot_general(
        scale, g_mat, (((1,), (0,)), ((), ())),
        preferred_element_type=jnp.float32)
    a_mat = e_mat * scale_row

    out_ref[...] = lax.dot_general(
        fv_ref[...], a_mat, (((1,), (0,)), ((), ())),
        preferred_element_type=jnp.float32) + name_row


def kernel(feature_values, name_table, W, b, name_indices):
    batch = feature_values.shape[0]
    b2 = b.reshape(1, _D_VAL)
    idx_pad = jnp.zeros((_FPAD,), jnp.int32).at[:_F].set(
        name_indices.astype(jnp.int32))
    table128 = jnp.pad(name_table, ((0, 0), (0, 128 - _D_NAME)))
    name_emb = _make_sc_gather()(table128, idx_pad)
    out = pl.pallas_call(
        _emb_kernel,
        grid=(batch // _BBLK,),
        in_specs=[
            pl.BlockSpec((_BBLK, _F), lambda i: (i, 0)),
            pl.BlockSpec((_FPAD, 128), lambda i: (0, 0)),
            pl.BlockSpec((_D_VAL, 1), lambda i: (0, 0)),
            pl.BlockSpec((1, _D_VAL), lambda i: (0, 0)),
        ],
        out_specs=pl.BlockSpec((_BBLK, _ROW), lambda i: (i, 0)),
        out_shape=jax.ShapeDtypeStruct((batch, _ROW), jnp.float32),
    )(feature_values, name_emb, W, b2)
    return out.reshape(batch, _F, _OUT_D)
